# baseline (speedup 1.0000x reference)
import functools

import jax
import jax.numpy as jnp
from jax import lax
from jax.experimental import pallas as pl
from jax.experimental.pallas import tpu as pltpu
from jax.experimental.pallas import tpu_sc as plsc

EPS = 1e-7
N = 10000
D = 256
H2 = 512
H4 = 1024
RB = 1000
NSTEP = N // RB

NE = 160000
CHUNK = 128
NCHUNK = 80
EPT = CHUNK * NCHUNK
NBUF = 2
NEP = EPT * 16
NP = 10112
STRIPE = NP // 16


def _ln(z, g, b):
    mu = jnp.mean(z, axis=-1, keepdims=True)
    var = jnp.mean((z - mu) ** 2, axis=-1, keepdims=True)
    return (z - mu) * lax.rsqrt(var + 1e-5) * g + b


def _tables(h, t):
    msg = jnp.maximum(h, 0.0) + EPS
    e = jnp.exp(msg * t)
    me = msg * e
    blocks = []
    for b in range(4):
        sl = slice(64 * b, 64 * (b + 1))
        blocks.append(jnp.concatenate([e[:, sl], me[:, sl]], axis=1)[None])
    return jnp.concatenate(blocks, axis=0)


def _fc_body(x_ref, w_ref, b_ref, t_ref, xn_ref, tab_ref):
    xn = jnp.maximum(
        jnp.dot(x_ref[...], w_ref[...], preferred_element_type=jnp.float32)
        + b_ref[...], 0.0)
    xn_ref[...] = xn
    tab_ref[...] = _tables(xn, t_ref[0, 0])


def _run_fc(x, w, b, t0):
    return pl.pallas_call(
        _fc_body,
        grid=(NSTEP,),
        in_specs=[
            pl.BlockSpec((RB, D), lambda i: (i, 0)),
            pl.BlockSpec((D, D), lambda i: (0, 0)),
            pl.BlockSpec((1, D), lambda i: (0, 0)),
            pl.BlockSpec(memory_space=pltpu.SMEM),
        ],
        out_specs=[
            pl.BlockSpec((RB, D), lambda i: (i, 0)),
            pl.BlockSpec((4, RB, 128), lambda i: (0, i, 0)),
        ],
        out_shape=[
            jax.ShapeDtypeStruct((N, D), jnp.float32),
            jax.ShapeDtypeStruct((4, N, 128), jnp.float32),
        ],
    )(x, w, b.reshape(1, D), t0.reshape(1, 1))


def _mlp_body(is_res, emit_tab, acc_ref, h_ref, w1_ref, b1_ref, g1_ref,
              be1_ref, w2_ref, b2_ref, lng_ref, lnb_ref, t_ref,
              hn_ref, tab_ref=None):
    h = h_ref[...]
    blk = acc_ref[...]
    aggr = jnp.concatenate(
        [blk[b, :, 64:128] / (blk[b, :, 0:64] + 1e-16) for b in range(4)],
        axis=1)
    out = aggr + h
    z = jnp.dot(out, w1_ref[...], preferred_element_type=jnp.float32) + b1_ref[...]
    z = jnp.maximum(_ln(z, g1_ref[...], be1_ref[...]), 0.0)
    z = jnp.dot(z, w2_ref[...], preferred_element_type=jnp.float32) + b2_ref[...]
    if is_res:
        hh = jnp.maximum(_ln(z, lng_ref[...], lnb_ref[...]), 0.0)
        hn = h + hh
    else:
        hn = z
    hn_ref[...] = hn
    if emit_tab:
        tab_ref[...] = _tables(hn, t_ref[0, 0])


def _run_mlp(acc, h, p, is_res, t_next):
    emit_tab = t_next is not None
    out_specs = [pl.BlockSpec((RB, D), lambda i: (i, 0))]
    out_shape = [jax.ShapeDtypeStruct((N, D), jnp.float32)]
    if emit_tab:
        out_specs.append(pl.BlockSpec((4, RB, 128), lambda i: (0, i, 0)))
        out_shape.append(jax.ShapeDtypeStruct((4, N, 128), jnp.float32))
    lng = p['ln_g']
    lnb = p['ln_b']
    t = t_next if emit_tab else jnp.float32(1.0)
    return pl.pallas_call(
        functools.partial(_mlp_body, is_res, emit_tab),
        grid=(NSTEP,),
        in_specs=[
            pl.BlockSpec((4, RB, 128), lambda i: (0, i, 0)),
            pl.BlockSpec((RB, D), lambda i: (i, 0)),
            pl.BlockSpec((D, H2), lambda i: (0, 0)),
            pl.BlockSpec((1, H2), lambda i: (0, 0)),
            pl.BlockSpec((1, H2), lambda i: (0, 0)),
            pl.BlockSpec((1, H2), lambda i: (0, 0)),
            pl.BlockSpec((H2, D), lambda i: (0, 0)),
            pl.BlockSpec((1, D), lambda i: (0, 0)),
            pl.BlockSpec((1, D), lambda i: (0, 0)),
            pl.BlockSpec((1, D), lambda i: (0, 0)),
            pl.BlockSpec(memory_space=pltpu.SMEM),
        ],
        out_specs=out_specs,
        out_shape=out_shape,
    )(acc, h, p['W1'], p['b1'].reshape(1, H2), p['g1'].reshape(1, H2),
      p['be1'].reshape(1, H2), p['W2'], p['b2'].reshape(1, D),
      lng.reshape(1, D), lnb.reshape(1, D), t.reshape(1, 1))


def _head_body(xn_ref, h1_ref, h2_ref, h3_ref, phi_ref, phib_ref,
               wa_ref, ba_ref, wb_ref, bb_ref, wc_ref, bc_ref,
               rho_ref, rhob_ref, cls_ref, clsb_ref, out_ref,
               num_ref, den_ref):
    i = pl.program_id(0)

    @pl.when(i == 0)
    def _init():
        num_ref[...] = jnp.zeros_like(num_ref)
        den_ref[...] = jnp.zeros_like(den_ref)

    phi = phi_ref[...]
    hp = (jnp.dot(xn_ref[...], phi[0:D], preferred_element_type=jnp.float32)
          + jnp.dot(h1_ref[...], phi[D:2 * D], preferred_element_type=jnp.float32)
          + jnp.dot(h2_ref[...], phi[2 * D:3 * D], preferred_element_type=jnp.float32)
          + jnp.dot(h3_ref[...], phi[3 * D:4 * D], preferred_element_type=jnp.float32)
          + phib_ref[...])
    hp = jnp.maximum(hp, 0.0)
    a = jnp.tanh(jnp.dot(hp, wa_ref[...], preferred_element_type=jnp.float32)
                 + ba_ref[...])
    b = jax.nn.sigmoid(jnp.dot(hp, wb_ref[...], preferred_element_type=jnp.float32)
                       + bb_ref[...])
    A = jnp.sum((a * b) * wc_ref[...], axis=1, keepdims=True) + bc_ref[0, 0]
    w = jnp.exp(A)
    num_ref[...] += jnp.sum(w * hp, axis=0, keepdims=True)
    den_ref[...] += jnp.sum(w)

    @pl.when(i == NSTEP - 1)
    def _final():
        pooled = num_ref[...] / den_ref[0, 0]
        hout = jnp.maximum(
            jnp.dot(pooled, rho_ref[...], preferred_element_type=jnp.float32)
            + rhob_ref[...], 0.0)
        logit = jnp.sum(hout * cls_ref[...]) + clsb_ref[0, 0]
        out_ref[...] = jnp.full((1, 1), logit, jnp.float32)


def _run_head(xn, h1, h2, h3, params):
    full = lambda shp: pl.BlockSpec(shp, lambda i: tuple(0 for _ in shp))
    rb = pl.BlockSpec((RB, D), lambda i: (i, 0))
    return pl.pallas_call(
        _head_body,
        grid=(NSTEP,),
        in_specs=[
            rb, rb, rb, rb,
            full((H4, H4)), full((1, H4)),
            full((H4, H4)), full((1, H4)),
            full((H4, H4)), full((1, H4)),
            full((1, H4)), full((1, 1)),
            full((H4, H4)), full((1, H4)),
            full((1, H4)), full((1, 1)),
        ],
        out_specs=pl.BlockSpec((1, 1), lambda i: (0, 0)),
        out_shape=jax.ShapeDtypeStruct((1, 1), jnp.float32),
        scratch_shapes=[
            pltpu.VMEM((1, H4), jnp.float32),
            pltpu.VMEM((1, 1), jnp.float32),
        ],
    )(xn, h1, h2, h3,
      params['phi_W'], params['phi_b'].reshape(1, H4),
      params['Wa'], params['ba'].reshape(1, H4),
      params['Wb'], params['bb'].reshape(1, H4),
      params['Wc'].reshape(1, H4), params['bc'].reshape(1, 1),
      params['rho_W'], params['rho_b'].reshape(1, H4),
      params['cls_W'].reshape(1, H4), params['cls_b'].reshape(1, 1))


NSUB = 2
SUBCH = NCHUNK // NSUB


def _agg_body(tab_ref, srcidx_ref, dstidx_ref, out_ref,
              acc_ref, srcslab_ref, dstslab_ref, gbuf_ref, gsem):
    c = lax.axis_index("c")
    s = lax.axis_index("s")
    base = s * STRIPE

    def _zero_stripe():
        def _z(k, _):
            gbuf_ref[0, k // 8, pl.ds((k % 8) * 16, 16)] = jnp.zeros(
                (16,), jnp.float32)
            return _
        lax.fori_loop(0, CHUNK * 8, _z, None)
        for k in range(STRIPE // CHUNK):
            pltpu.sync_copy(gbuf_ref.at[0],
                            acc_ref.at[pl.ds(base + CHUNK * k, CHUNK)])
        rem = STRIPE % CHUNK
        pltpu.sync_copy(gbuf_ref.at[0, pl.ds(0, rem)],
                        acc_ref.at[pl.ds(base + STRIPE - rem, rem)])

    _zero_stripe()
    plsc.subcore_barrier()

    for bi in range(2):
        b = 2 * c + bi
        for sub in range(NSUB):
            h0 = sub * SUBCH
            pltpu.sync_copy(srcidx_ref.at[b, s, pl.ds(h0, SUBCH)],
                            srcslab_ref)
            pltpu.sync_copy(dstidx_ref.at[s, pl.ds(h0, SUBCH)],
                            dstslab_ref)
            for k in range(NBUF - 1):
                pltpu.async_copy(tab_ref.at[srcslab_ref.at[k]],
                                 gbuf_ref.at[k], gsem)

            def _chunk(k, _):
                j = lax.rem(k, NBUF)

                @pl.when(k < SUBCH - (NBUF - 1))
                def _():
                    pltpu.async_copy(
                        tab_ref.at[srcslab_ref.at[k + NBUF - 1]],
                        gbuf_ref.at[lax.rem(k + NBUF - 1, NBUF)], gsem)
                pltpu.make_async_copy(tab_ref.at[pl.ds(0, CHUNK)],
                                      gbuf_ref.at[j], gsem).wait()
                pltpu.sync_copy(gbuf_ref.at[j],
                                acc_ref.at[dstslab_ref.at[k]], add=True)
                return _
            lax.fori_loop(0, SUBCH, _chunk, None)
        plsc.subcore_barrier()

        pltpu.sync_copy(acc_ref.at[pl.ds(base, STRIPE)],
                        out_ref.at[b, pl.ds(base, STRIPE)])
        if bi == 0:
            _zero_stripe()
            plsc.subcore_barrier()


def _run_agg(tabflat, srcidx, dstidx):
    mesh = plsc.VectorSubcoreMesh(core_axis_name="c", subcore_axis_name="s")
    return pl.kernel(
        _agg_body,
        out_type=jax.ShapeDtypeStruct((4, NP, 128), jnp.float32),
        mesh=mesh,
        scratch_types=[
            pltpu.VMEM_SHARED((NP, 128), jnp.float32),
            pltpu.VMEM((SUBCH, CHUNK), jnp.int32),
            pltpu.VMEM((SUBCH, CHUNK), jnp.int32),
            pltpu.VMEM((NBUF, CHUNK, 128), jnp.float32),
            pltpu.SemaphoreType.DMA,
        ],
    )(tabflat, srcidx, dstidx)


def _edge_slabs(edge_index):
    src = edge_index[0].astype(jnp.int32)
    dst = edge_index[1].astype(jnp.int32)
    src_p = jnp.concatenate([src, jnp.zeros((NEP - NE,), jnp.int32)])
    dst_p = jnp.concatenate([dst, jnp.full((NEP - NE,), N, jnp.int32)])
    offs = (jnp.arange(4, dtype=jnp.int32) * N)[:, None]
    srcidx = (src_p[None, :] + offs).reshape(4, 16, NCHUNK, CHUNK)
    dstidx = dst_p.reshape(16, NCHUNK, CHUNK)
    return srcidx, dstidx


def kernel(fea_old, x, edge_index, edge_attr, params):
    L = params['layers']
    srcidx, dstidx = _edge_slabs(edge_index)
    xn, tab = _run_fc(x, params['fc_W'], params['fc_b'], L[0]['t'])
    acc = _run_agg(tab.reshape(4 * N, 128), srcidx, dstidx)[:, :N, :]
    h1, tab = _run_mlp(acc, xn, L[0], False, L[1]['t'])
    acc = _run_agg(tab.reshape(4 * N, 128), srcidx, dstidx)[:, :N, :]
    h2, tab = _run_mlp(acc, h1, L[1], True, L[2]['t'])
    acc = _run_agg(tab.reshape(4 * N, 128), srcidx, dstidx)[:, :N, :]
    (h3,) = _run_mlp(acc, h2, L[2], True, None)
    return _run_head(xn, h1, h2, h3, params)


# speedup vs baseline: 2.1871x; 2.1871x over previous
import functools

import jax
import jax.numpy as jnp
from jax import lax
from jax.experimental import pallas as pl
from jax.experimental.pallas import tpu as pltpu
from jax.experimental.pallas import tpu_sc as plsc

EPS = 1e-7
N = 10000
D = 256
H2 = 512
H4 = 1024
RB = 1000
NSTEP = N // RB

NE = 160000
CHUNK = 128
NCHUNK = 80
EPT = CHUNK * NCHUNK
NBUF = 2
NEP = EPT * 16
NP = 10112
STRIPE = NP // 16


def _ln(z, g, b):
    mu = jnp.mean(z, axis=-1, keepdims=True)
    var = jnp.mean((z - mu) ** 2, axis=-1, keepdims=True)
    return (z - mu) * lax.rsqrt(var + 1e-5) * g + b


def _tables(h, t):
    msg = jnp.maximum(h, 0.0) + EPS
    e = jnp.exp(msg * t)
    me = msg * e
    blocks = []
    for b in range(4):
        sl = slice(64 * b, 64 * (b + 1))
        blocks.append(jnp.concatenate([e[:, sl], me[:, sl]], axis=1)[None])
    return jnp.concatenate(blocks, axis=0)


def _fc_body(x_ref, w_ref, b_ref, t_ref, xn_ref, tab_ref):
    xn = jnp.maximum(
        jnp.dot(x_ref[...], w_ref[...], preferred_element_type=jnp.float32)
        + b_ref[...], 0.0)
    xn_ref[...] = xn
    tab_ref[...] = _tables(xn, t_ref[0, 0])


def _run_fc(x, w, b, t0):
    return pl.pallas_call(
        _fc_body,
        grid=(NSTEP,),
        in_specs=[
            pl.BlockSpec((RB, D), lambda i: (i, 0)),
            pl.BlockSpec((D, D), lambda i: (0, 0)),
            pl.BlockSpec((1, D), lambda i: (0, 0)),
            pl.BlockSpec(memory_space=pltpu.SMEM),
        ],
        out_specs=[
            pl.BlockSpec((RB, D), lambda i: (i, 0)),
            pl.BlockSpec((4, RB, 128), lambda i: (0, i, 0)),
        ],
        out_shape=[
            jax.ShapeDtypeStruct((N, D), jnp.float32),
            jax.ShapeDtypeStruct((4, N, 128), jnp.float32),
        ],
    )(x, w, b.reshape(1, D), t0.reshape(1, 1))


def _mlp_body(is_res, emit_tab, acc_ref, h_ref, w1_ref, b1_ref, g1_ref,
              be1_ref, w2_ref, b2_ref, lng_ref, lnb_ref, t_ref,
              hn_ref, tab_ref=None):
    h = h_ref[...]
    blk = acc_ref[...]
    aggr = jnp.concatenate(
        [blk[b, :, 64:128] / (blk[b, :, 0:64] + 1e-16) for b in range(4)],
        axis=1)
    out = aggr + h
    z = jnp.dot(out, w1_ref[...], preferred_element_type=jnp.float32) + b1_ref[...]
    z = jnp.maximum(_ln(z, g1_ref[...], be1_ref[...]), 0.0)
    z = jnp.dot(z, w2_ref[...], preferred_element_type=jnp.float32) + b2_ref[...]
    if is_res:
        hh = jnp.maximum(_ln(z, lng_ref[...], lnb_ref[...]), 0.0)
        hn = h + hh
    else:
        hn = z
    hn_ref[...] = hn
    if emit_tab:
        tab_ref[...] = _tables(hn, t_ref[0, 0])


def _run_mlp(acc, h, p, is_res, t_next):
    emit_tab = t_next is not None
    out_specs = [pl.BlockSpec((RB, D), lambda i: (i, 0))]
    out_shape = [jax.ShapeDtypeStruct((N, D), jnp.float32)]
    if emit_tab:
        out_specs.append(pl.BlockSpec((4, RB, 128), lambda i: (0, i, 0)))
        out_shape.append(jax.ShapeDtypeStruct((4, N, 128), jnp.float32))
    lng = p['ln_g']
    lnb = p['ln_b']
    t = t_next if emit_tab else jnp.float32(1.0)
    return pl.pallas_call(
        functools.partial(_mlp_body, is_res, emit_tab),
        grid=(NSTEP,),
        in_specs=[
            pl.BlockSpec((4, RB, 128), lambda i: (0, i, 0)),
            pl.BlockSpec((RB, D), lambda i: (i, 0)),
            pl.BlockSpec((D, H2), lambda i: (0, 0)),
            pl.BlockSpec((1, H2), lambda i: (0, 0)),
            pl.BlockSpec((1, H2), lambda i: (0, 0)),
            pl.BlockSpec((1, H2), lambda i: (0, 0)),
            pl.BlockSpec((H2, D), lambda i: (0, 0)),
            pl.BlockSpec((1, D), lambda i: (0, 0)),
            pl.BlockSpec((1, D), lambda i: (0, 0)),
            pl.BlockSpec((1, D), lambda i: (0, 0)),
            pl.BlockSpec(memory_space=pltpu.SMEM),
        ],
        out_specs=out_specs,
        out_shape=out_shape,
    )(acc, h, p['W1'], p['b1'].reshape(1, H2), p['g1'].reshape(1, H2),
      p['be1'].reshape(1, H2), p['W2'], p['b2'].reshape(1, D),
      lng.reshape(1, D), lnb.reshape(1, D), t.reshape(1, 1))


def _head_body(xn_ref, h1_ref, h2_ref, h3_ref, phi_ref, phib_ref,
               wa_ref, ba_ref, wb_ref, bb_ref, wc_ref, bc_ref,
               rho_ref, rhob_ref, cls_ref, clsb_ref, out_ref,
               num_ref, den_ref):
    i = pl.program_id(0)

    @pl.when(i == 0)
    def _init():
        num_ref[...] = jnp.zeros_like(num_ref)
        den_ref[...] = jnp.zeros_like(den_ref)

    phi = phi_ref[...]
    hp = (jnp.dot(xn_ref[...], phi[0:D], preferred_element_type=jnp.float32)
          + jnp.dot(h1_ref[...], phi[D:2 * D], preferred_element_type=jnp.float32)
          + jnp.dot(h2_ref[...], phi[2 * D:3 * D], preferred_element_type=jnp.float32)
          + jnp.dot(h3_ref[...], phi[3 * D:4 * D], preferred_element_type=jnp.float32)
          + phib_ref[...])
    hp = jnp.maximum(hp, 0.0)
    a = jnp.tanh(jnp.dot(hp, wa_ref[...], preferred_element_type=jnp.float32)
                 + ba_ref[...])
    b = jax.nn.sigmoid(jnp.dot(hp, wb_ref[...], preferred_element_type=jnp.float32)
                       + bb_ref[...])
    A = jnp.sum((a * b) * wc_ref[...], axis=1, keepdims=True) + bc_ref[0, 0]
    w = jnp.exp(A)
    num_ref[...] += jnp.sum(w * hp, axis=0, keepdims=True)
    den_ref[...] += jnp.sum(w)

    @pl.when(i == NSTEP - 1)
    def _final():
        pooled = num_ref[...] / den_ref[0, 0]
        hout = jnp.maximum(
            jnp.dot(pooled, rho_ref[...], preferred_element_type=jnp.float32)
            + rhob_ref[...], 0.0)
        logit = jnp.sum(hout * cls_ref[...]) + clsb_ref[0, 0]
        out_ref[...] = jnp.full((1, 1), logit, jnp.float32)


def _run_head(xn, h1, h2, h3, params):
    full = lambda shp: pl.BlockSpec(shp, lambda i: tuple(0 for _ in shp))
    rb = pl.BlockSpec((RB, D), lambda i: (i, 0))
    return pl.pallas_call(
        _head_body,
        grid=(NSTEP,),
        in_specs=[
            rb, rb, rb, rb,
            full((H4, H4)), full((1, H4)),
            full((H4, H4)), full((1, H4)),
            full((H4, H4)), full((1, H4)),
            full((1, H4)), full((1, 1)),
            full((H4, H4)), full((1, H4)),
            full((1, H4)), full((1, 1)),
        ],
        out_specs=pl.BlockSpec((1, 1), lambda i: (0, 0)),
        out_shape=jax.ShapeDtypeStruct((1, 1), jnp.float32),
        scratch_shapes=[
            pltpu.VMEM((1, H4), jnp.float32),
            pltpu.VMEM((1, 1), jnp.float32),
        ],
    )(xn, h1, h2, h3,
      params['phi_W'], params['phi_b'].reshape(1, H4),
      params['Wa'], params['ba'].reshape(1, H4),
      params['Wb'], params['bb'].reshape(1, H4),
      params['Wc'].reshape(1, H4), params['bc'].reshape(1, 1),
      params['rho_W'], params['rho_b'].reshape(1, H4),
      params['cls_W'].reshape(1, H4), params['cls_b'].reshape(1, 1))


NSUB = 2
SUBCH = NCHUNK // NSUB


def _agg_body(tab_ref, srcidx_ref, dstidx_ref, out_ref,
              acc_ref, srcslab_ref, dstslab_ref, gbuf_ref, gsem):
    c = lax.axis_index("c")
    s = lax.axis_index("s")
    base = s * STRIPE

    def _zero_stripe():
        def _z(k, _):
            gbuf_ref[0, k // 8, pl.ds((k % 8) * 16, 16)] = jnp.zeros(
                (16,), jnp.float32)
            return _
        lax.fori_loop(0, CHUNK * 8, _z, None)
        for k in range(STRIPE // CHUNK):
            pltpu.sync_copy(gbuf_ref.at[0],
                            acc_ref.at[pl.ds(base + CHUNK * k, CHUNK)])
        rem = STRIPE % CHUNK
        pltpu.sync_copy(gbuf_ref.at[0, pl.ds(0, rem)],
                        acc_ref.at[pl.ds(base + STRIPE - rem, rem)])

    _zero_stripe()
    plsc.subcore_barrier()

    for bi in range(2):
        b = 2 * c + bi
        for sub in range(NSUB):
            h0 = sub * SUBCH
            pltpu.sync_copy(srcidx_ref.at[b, s, pl.ds(h0, SUBCH)],
                            srcslab_ref)
            pltpu.sync_copy(dstidx_ref.at[s, pl.ds(h0, SUBCH)],
                            dstslab_ref)
            for k in range(NBUF - 1):
                pltpu.async_copy(tab_ref.at[srcslab_ref.at[k]],
                                 gbuf_ref.at[k], gsem)

            def _chunk(k, _):
                j = lax.rem(k, NBUF)

                @pl.when(k < SUBCH - (NBUF - 1))
                def _():
                    pltpu.async_copy(
                        tab_ref.at[srcslab_ref.at[k + NBUF - 1]],
                        gbuf_ref.at[lax.rem(k + NBUF - 1, NBUF)], gsem)
                pltpu.make_async_copy(tab_ref.at[pl.ds(0, CHUNK)],
                                      gbuf_ref.at[j], gsem).wait()
                pltpu.sync_copy(gbuf_ref.at[j],
                                acc_ref.at[dstslab_ref.at[k]], add=True)
                return _
            lax.fori_loop(0, SUBCH, _chunk, None)
        plsc.subcore_barrier()

        pltpu.sync_copy(acc_ref.at[pl.ds(base, STRIPE)],
                        out_ref.at[b, pl.ds(base, STRIPE)])
        if bi == 0:
            _zero_stripe()
            plsc.subcore_barrier()


def _run_agg(tabflat, srcidx, dstidx):
    mesh = plsc.VectorSubcoreMesh(core_axis_name="c", subcore_axis_name="s")
    return pl.kernel(
        _agg_body,
        out_type=jax.ShapeDtypeStruct((4, NP, 128), jnp.float32),
        mesh=mesh,
        scratch_types=[
            pltpu.VMEM_SHARED((NP, 128), jnp.float32),
            pltpu.VMEM((SUBCH, CHUNK), jnp.int32),
            pltpu.VMEM((SUBCH, CHUNK), jnp.int32),
            pltpu.VMEM((NBUF, CHUNK, 128), jnp.float32),
            pltpu.SemaphoreType.DMA,
        ],
    )(tabflat, srcidx, dstidx)


PAD_PER = EPT - NE // 16


def _edge_slabs(edge_index):
    src = edge_index[0].astype(jnp.int32).reshape(16, NE // 16)
    dst = edge_index[1].astype(jnp.int32).reshape(16, NE // 16)
    pad_src = jnp.broadcast_to(jnp.arange(PAD_PER, dtype=jnp.int32) % 997,
                               (16, PAD_PER))
    pad_dst = jnp.broadcast_to(
        N + jnp.arange(PAD_PER, dtype=jnp.int32) % (NP - N), (16, PAD_PER))
    src_p = jnp.concatenate([src, pad_src], axis=1)
    dst_p = jnp.concatenate([dst, pad_dst], axis=1)
    offs = (jnp.arange(4, dtype=jnp.int32) * N)[:, None, None]
    srcidx = (src_p[None] + offs).reshape(4, 16, NCHUNK, CHUNK)
    dstidx = dst_p.reshape(16, NCHUNK, CHUNK)
    return srcidx, dstidx


def kernel(fea_old, x, edge_index, edge_attr, params):
    L = params['layers']
    srcidx, dstidx = _edge_slabs(edge_index)
    xn, tab = _run_fc(x, params['fc_W'], params['fc_b'], L[0]['t'])
    acc = _run_agg(tab.reshape(4 * N, 128), srcidx, dstidx)[:, :N, :]
    h1, tab = _run_mlp(acc, xn, L[0], False, L[1]['t'])
    acc = _run_agg(tab.reshape(4 * N, 128), srcidx, dstidx)[:, :N, :]
    h2, tab = _run_mlp(acc, h1, L[1], True, L[2]['t'])
    acc = _run_agg(tab.reshape(4 * N, 128), srcidx, dstidx)[:, :N, :]
    (h3,) = _run_mlp(acc, h2, L[2], True, None)
    return _run_head(xn, h1, h2, h3, params)


# speedup vs baseline: 2.1903x; 1.0015x over previous
import functools

import jax
import jax.numpy as jnp
from jax import lax
from jax.experimental import pallas as pl
from jax.experimental.pallas import tpu as pltpu
from jax.experimental.pallas import tpu_sc as plsc

EPS = 1e-7
N = 10000
D = 256
H2 = 512
H4 = 1024
RB = 1000
NSTEP = N // RB

NE = 160000
CHUNK = 128
NCHUNK = 79
EPT = CHUNK * NCHUNK
NBUF = 2
NEP = EPT * 16
NP = 10112
STRIPE = NP // 16


def _bdot(x, w):
    return jnp.dot(x.astype(jnp.bfloat16), w.astype(jnp.bfloat16),
                   preferred_element_type=jnp.float32)


def _ln(z, g, b):
    mu = jnp.mean(z, axis=-1, keepdims=True)
    var = jnp.mean((z - mu) ** 2, axis=-1, keepdims=True)
    return (z - mu) * lax.rsqrt(var + 1e-5) * g + b


def _tables(h, t):
    msg = jnp.maximum(h, 0.0) + EPS
    e = jnp.exp(msg * t)
    me = msg * e
    blocks = []
    for b in range(4):
        sl = slice(64 * b, 64 * (b + 1))
        blocks.append(jnp.concatenate([e[:, sl], me[:, sl]], axis=1)[None])
    return jnp.concatenate(blocks, axis=0)


def _fc_body(x_ref, w_ref, b_ref, t_ref, xn_ref, tab_ref):
    xn = jnp.maximum(
        _bdot(x_ref[...], w_ref[...])
        + b_ref[...], 0.0)
    xn_ref[...] = xn
    tab_ref[...] = _tables(xn, t_ref[0, 0])


def _run_fc(x, w, b, t0):
    return pl.pallas_call(
        _fc_body,
        grid=(NSTEP,),
        in_specs=[
            pl.BlockSpec((RB, D), lambda i: (i, 0)),
            pl.BlockSpec((D, D), lambda i: (0, 0)),
            pl.BlockSpec((1, D), lambda i: (0, 0)),
            pl.BlockSpec(memory_space=pltpu.SMEM),
        ],
        out_specs=[
            pl.BlockSpec((RB, D), lambda i: (i, 0)),
            pl.BlockSpec((4, RB, 128), lambda i: (0, i, 0)),
        ],
        out_shape=[
            jax.ShapeDtypeStruct((N, D), jnp.float32),
            jax.ShapeDtypeStruct((4, N, 128), jnp.float32),
        ],
    )(x, w, b.reshape(1, D), t0.reshape(1, 1))


def _mlp_body(is_res, emit_tab, acc_ref, h_ref, w1_ref, b1_ref, g1_ref,
              be1_ref, w2_ref, b2_ref, lng_ref, lnb_ref, t_ref,
              hn_ref, tab_ref=None):
    h = h_ref[...]
    blk = acc_ref[...]
    aggr = jnp.concatenate(
        [blk[b, :, 64:128] / (blk[b, :, 0:64] + 1e-16) for b in range(4)],
        axis=1)
    out = aggr + h
    z = _bdot(out, w1_ref[...]) + b1_ref[...]
    z = jnp.maximum(_ln(z, g1_ref[...], be1_ref[...]), 0.0)
    z = _bdot(z, w2_ref[...]) + b2_ref[...]
    if is_res:
        hh = jnp.maximum(_ln(z, lng_ref[...], lnb_ref[...]), 0.0)
        hn = h + hh
    else:
        hn = z
    hn_ref[...] = hn
    if emit_tab:
        tab_ref[...] = _tables(hn, t_ref[0, 0])


def _run_mlp(acc, h, p, is_res, t_next):
    emit_tab = t_next is not None
    out_specs = [pl.BlockSpec((RB, D), lambda i: (i, 0))]
    out_shape = [jax.ShapeDtypeStruct((N, D), jnp.float32)]
    if emit_tab:
        out_specs.append(pl.BlockSpec((4, RB, 128), lambda i: (0, i, 0)))
        out_shape.append(jax.ShapeDtypeStruct((4, N, 128), jnp.float32))
    lng = p['ln_g']
    lnb = p['ln_b']
    t = t_next if emit_tab else jnp.float32(1.0)
    return pl.pallas_call(
        functools.partial(_mlp_body, is_res, emit_tab),
        grid=(NSTEP,),
        in_specs=[
            pl.BlockSpec((4, RB, 128), lambda i: (0, i, 0)),
            pl.BlockSpec((RB, D), lambda i: (i, 0)),
            pl.BlockSpec((D, H2), lambda i: (0, 0)),
            pl.BlockSpec((1, H2), lambda i: (0, 0)),
            pl.BlockSpec((1, H2), lambda i: (0, 0)),
            pl.BlockSpec((1, H2), lambda i: (0, 0)),
            pl.BlockSpec((H2, D), lambda i: (0, 0)),
            pl.BlockSpec((1, D), lambda i: (0, 0)),
            pl.BlockSpec((1, D), lambda i: (0, 0)),
            pl.BlockSpec((1, D), lambda i: (0, 0)),
            pl.BlockSpec(memory_space=pltpu.SMEM),
        ],
        out_specs=out_specs,
        out_shape=out_shape,
    )(acc, h, p['W1'], p['b1'].reshape(1, H2), p['g1'].reshape(1, H2),
      p['be1'].reshape(1, H2), p['W2'], p['b2'].reshape(1, D),
      lng.reshape(1, D), lnb.reshape(1, D), t.reshape(1, 1))


def _head_body(xn_ref, h1_ref, h2_ref, h3_ref, phi_ref, phib_ref,
               wa_ref, ba_ref, wb_ref, bb_ref, wc_ref, bc_ref,
               rho_ref, rhob_ref, cls_ref, clsb_ref, out_ref,
               num_ref, den_ref):
    i = pl.program_id(0)

    @pl.when(i == 0)
    def _init():
        num_ref[...] = jnp.zeros_like(num_ref)
        den_ref[...] = jnp.zeros_like(den_ref)

    phi = phi_ref[...]
    hp = (_bdot(xn_ref[...], phi[0:D])
          + _bdot(h1_ref[...], phi[D:2 * D])
          + _bdot(h2_ref[...], phi[2 * D:3 * D])
          + _bdot(h3_ref[...], phi[3 * D:4 * D])
          + phib_ref[...])
    hp = jnp.maximum(hp, 0.0)
    a = jnp.tanh(_bdot(hp, wa_ref[...])
                 + ba_ref[...])
    b = jax.nn.sigmoid(_bdot(hp, wb_ref[...])
                       + bb_ref[...])
    A = jnp.sum((a * b) * wc_ref[...], axis=1, keepdims=True) + bc_ref[0, 0]
    w = jnp.exp(A)
    num_ref[...] += jnp.sum(w * hp, axis=0, keepdims=True)
    den_ref[...] += jnp.sum(w)

    @pl.when(i == NSTEP - 1)
    def _final():
        pooled = num_ref[...] / den_ref[0, 0]
        hout = jnp.maximum(
            _bdot(pooled, rho_ref[...])
            + rhob_ref[...], 0.0)
        logit = jnp.sum(hout * cls_ref[...]) + clsb_ref[0, 0]
        out_ref[...] = jnp.full((1, 1), logit, jnp.float32)


def _run_head(xn, h1, h2, h3, params):
    full = lambda shp: pl.BlockSpec(shp, lambda i: tuple(0 for _ in shp))
    rb = pl.BlockSpec((RB, D), lambda i: (i, 0))
    return pl.pallas_call(
        _head_body,
        grid=(NSTEP,),
        in_specs=[
            rb, rb, rb, rb,
            full((H4, H4)), full((1, H4)),
            full((H4, H4)), full((1, H4)),
            full((H4, H4)), full((1, H4)),
            full((1, H4)), full((1, 1)),
            full((H4, H4)), full((1, H4)),
            full((1, H4)), full((1, 1)),
        ],
        out_specs=pl.BlockSpec((1, 1), lambda i: (0, 0)),
        out_shape=jax.ShapeDtypeStruct((1, 1), jnp.float32),
        scratch_shapes=[
            pltpu.VMEM((1, H4), jnp.float32),
            pltpu.VMEM((1, 1), jnp.float32),
        ],
    )(xn, h1, h2, h3,
      params['phi_W'], params['phi_b'].reshape(1, H4),
      params['Wa'], params['ba'].reshape(1, H4),
      params['Wb'], params['bb'].reshape(1, H4),
      params['Wc'].reshape(1, H4), params['bc'].reshape(1, 1),
      params['rho_W'], params['rho_b'].reshape(1, H4),
      params['cls_W'].reshape(1, H4), params['cls_b'].reshape(1, 1))


SUBS = (40, 39)


def _agg_body(tab_ref, srcidx_ref, dstidx_ref, out_ref,
              acc_ref, srcslab_ref, dstslab_ref, gbuf_ref, gsem):
    c = lax.axis_index("c")
    s = lax.axis_index("s")
    base = s * STRIPE

    def _zero_stripe():
        def _z(k, _):
            gbuf_ref[0, k // 8, pl.ds((k % 8) * 16, 16)] = jnp.zeros(
                (16,), jnp.float32)
            return _
        lax.fori_loop(0, CHUNK * 8, _z, None)
        for k in range(STRIPE // CHUNK):
            pltpu.sync_copy(gbuf_ref.at[0],
                            acc_ref.at[pl.ds(base + CHUNK * k, CHUNK)])
        rem = STRIPE % CHUNK
        pltpu.sync_copy(gbuf_ref.at[0, pl.ds(0, rem)],
                        acc_ref.at[pl.ds(base + STRIPE - rem, rem)])

    _zero_stripe()
    plsc.subcore_barrier()

    for bi in range(2):
        b = 2 * c + bi
        for sub, nch in enumerate(SUBS):
            h0 = sub * SUBS[0]
            pltpu.sync_copy(srcidx_ref.at[b, s, pl.ds(h0, nch)],
                            srcslab_ref.at[pl.ds(0, nch)])
            pltpu.sync_copy(dstidx_ref.at[s, pl.ds(h0, nch)],
                            dstslab_ref.at[pl.ds(0, nch)])
            for k in range(NBUF - 1):
                pltpu.async_copy(tab_ref.at[srcslab_ref.at[k]],
                                 gbuf_ref.at[k], gsem)

            def _chunk(k, _):
                j = lax.rem(k, NBUF)

                @pl.when(k < nch - (NBUF - 1))
                def _():
                    pltpu.async_copy(
                        tab_ref.at[srcslab_ref.at[k + NBUF - 1]],
                        gbuf_ref.at[lax.rem(k + NBUF - 1, NBUF)], gsem)
                pltpu.make_async_copy(tab_ref.at[pl.ds(0, CHUNK)],
                                      gbuf_ref.at[j], gsem).wait()
                pltpu.sync_copy(gbuf_ref.at[j],
                                acc_ref.at[dstslab_ref.at[k]], add=True)
                return _
            lax.fori_loop(0, nch, _chunk, None)
        plsc.subcore_barrier()

        pltpu.sync_copy(acc_ref.at[pl.ds(base, STRIPE)],
                        out_ref.at[b, pl.ds(base, STRIPE)])
        if bi == 0:
            _zero_stripe()
            plsc.subcore_barrier()


def _run_agg(tabflat, srcidx, dstidx):
    mesh = plsc.VectorSubcoreMesh(core_axis_name="c", subcore_axis_name="s")
    return pl.kernel(
        _agg_body,
        out_type=jax.ShapeDtypeStruct((4, NP, 128), jnp.float32),
        mesh=mesh,
        scratch_types=[
            pltpu.VMEM_SHARED((NP, 128), jnp.float32),
            pltpu.VMEM((SUBS[0], CHUNK), jnp.int32),
            pltpu.VMEM((SUBS[0], CHUNK), jnp.int32),
            pltpu.VMEM((NBUF, CHUNK, 128), jnp.float32),
            pltpu.SemaphoreType.DMA,
        ],
    )(tabflat, srcidx, dstidx)


PAD_PER = EPT - NE // 16


def _edge_slabs(edge_index):
    src = edge_index[0].astype(jnp.int32).reshape(16, NE // 16)
    dst = edge_index[1].astype(jnp.int32).reshape(16, NE // 16)
    pad_src = jnp.broadcast_to(jnp.arange(PAD_PER, dtype=jnp.int32) % 997,
                               (16, PAD_PER))
    pad_dst = jnp.broadcast_to(
        N + jnp.arange(PAD_PER, dtype=jnp.int32) % (NP - N), (16, PAD_PER))
    src_p = jnp.concatenate([src, pad_src], axis=1)
    dst_p = jnp.concatenate([dst, pad_dst], axis=1)
    offs = (jnp.arange(4, dtype=jnp.int32) * N)[:, None, None]
    srcidx = (src_p[None] + offs).reshape(4, 16, NCHUNK, CHUNK)
    dstidx = dst_p.reshape(16, NCHUNK, CHUNK)
    return srcidx, dstidx


def kernel(fea_old, x, edge_index, edge_attr, params):
    L = params['layers']
    srcidx, dstidx = _edge_slabs(edge_index)
    xn, tab = _run_fc(x, params['fc_W'], params['fc_b'], L[0]['t'])
    acc = _run_agg(tab.reshape(4 * N, 128), srcidx, dstidx)[:, :N, :]
    h1, tab = _run_mlp(acc, xn, L[0], False, L[1]['t'])
    acc = _run_agg(tab.reshape(4 * N, 128), srcidx, dstidx)[:, :N, :]
    h2, tab = _run_mlp(acc, h1, L[1], True, L[2]['t'])
    acc = _run_agg(tab.reshape(4 * N, 128), srcidx, dstidx)[:, :N, :]
    (h3,) = _run_mlp(acc, h2, L[2], True, None)
    return _run_head(xn, h1, h2, h3, params)


# speedup vs baseline: 2.2035x; 1.0060x over previous
import functools

import jax
import jax.numpy as jnp
from jax import lax
from jax.experimental import pallas as pl
from jax.experimental.pallas import tpu as pltpu
from jax.experimental.pallas import tpu_sc as plsc

EPS = 1e-7
N = 10000
D = 256
H2 = 512
H4 = 1024
RB = 1000
NSTEP = N // RB

NE = 160000
CHUNK = 128
NCHUNK = 79
EPT = CHUNK * NCHUNK
NBUF = 2
NEP = EPT * 16
NP = 10112
STRIPE = NP // 16


def _bdot(x, w):
    return jnp.dot(x, w, preferred_element_type=jnp.float32)


def _ln(z, g, b):
    mu = jnp.mean(z, axis=-1, keepdims=True)
    var = jnp.mean((z - mu) ** 2, axis=-1, keepdims=True)
    return (z - mu) * lax.rsqrt(var + 1e-5) * g + b


def _tables(h, t):
    msg = jnp.maximum(h, 0.0) + EPS
    e = jnp.exp(msg * t)
    me = msg * e
    blocks = []
    for b in range(4):
        sl = slice(64 * b, 64 * (b + 1))
        blocks.append(jnp.concatenate([e[:, sl], me[:, sl]], axis=1)[None])
    return jnp.concatenate(blocks, axis=0)


def _fc_body(x_ref, w_ref, b_ref, t_ref, xn_ref, tab_ref):
    xn = jnp.maximum(
        _bdot(x_ref[...], w_ref[...])
        + b_ref[...], 0.0)
    xn_ref[...] = xn
    tab_ref[...] = _tables(xn, t_ref[0, 0])


def _run_fc(x, w, b, t0):
    return pl.pallas_call(
        _fc_body,
        grid=(NSTEP,),
        in_specs=[
            pl.BlockSpec((RB, D), lambda i: (i, 0)),
            pl.BlockSpec((D, D), lambda i: (0, 0)),
            pl.BlockSpec((1, D), lambda i: (0, 0)),
            pl.BlockSpec(memory_space=pltpu.SMEM),
        ],
        out_specs=[
            pl.BlockSpec((RB, D), lambda i: (i, 0)),
            pl.BlockSpec((4, RB, 128), lambda i: (0, i, 0)),
        ],
        out_shape=[
            jax.ShapeDtypeStruct((N, D), jnp.float32),
            jax.ShapeDtypeStruct((4, N, 128), jnp.float32),
        ],
    )(x, w, b.reshape(1, D), t0.reshape(1, 1))


def _mlp_body(is_res, emit_tab, acc_ref, h_ref, w1_ref, b1_ref, g1_ref,
              be1_ref, w2_ref, b2_ref, lng_ref, lnb_ref, t_ref,
              hn_ref, tab_ref=None):
    h = h_ref[...]
    blk = acc_ref[...]
    aggr = jnp.concatenate(
        [blk[b, :, 64:128] / (blk[b, :, 0:64] + 1e-16) for b in range(4)],
        axis=1)
    out = aggr + h
    z = _bdot(out, w1_ref[...]) + b1_ref[...]
    z = jnp.maximum(_ln(z, g1_ref[...], be1_ref[...]), 0.0)
    z = _bdot(z, w2_ref[...]) + b2_ref[...]
    if is_res:
        hh = jnp.maximum(_ln(z, lng_ref[...], lnb_ref[...]), 0.0)
        hn = h + hh
    else:
        hn = z
    hn_ref[...] = hn
    if emit_tab:
        tab_ref[...] = _tables(hn, t_ref[0, 0])


def _run_mlp(acc, h, p, is_res, t_next):
    emit_tab = t_next is not None
    out_specs = [pl.BlockSpec((RB, D), lambda i: (i, 0))]
    out_shape = [jax.ShapeDtypeStruct((N, D), jnp.float32)]
    if emit_tab:
        out_specs.append(pl.BlockSpec((4, RB, 128), lambda i: (0, i, 0)))
        out_shape.append(jax.ShapeDtypeStruct((4, N, 128), jnp.float32))
    lng = p['ln_g']
    lnb = p['ln_b']
    t = t_next if emit_tab else jnp.float32(1.0)
    return pl.pallas_call(
        functools.partial(_mlp_body, is_res, emit_tab),
        grid=(NSTEP,),
        in_specs=[
            pl.BlockSpec((4, RB, 128), lambda i: (0, i, 0)),
            pl.BlockSpec((RB, D), lambda i: (i, 0)),
            pl.BlockSpec((D, H2), lambda i: (0, 0)),
            pl.BlockSpec((1, H2), lambda i: (0, 0)),
            pl.BlockSpec((1, H2), lambda i: (0, 0)),
            pl.BlockSpec((1, H2), lambda i: (0, 0)),
            pl.BlockSpec((H2, D), lambda i: (0, 0)),
            pl.BlockSpec((1, D), lambda i: (0, 0)),
            pl.BlockSpec((1, D), lambda i: (0, 0)),
            pl.BlockSpec((1, D), lambda i: (0, 0)),
            pl.BlockSpec(memory_space=pltpu.SMEM),
        ],
        out_specs=out_specs,
        out_shape=out_shape,
    )(acc, h, p['W1'], p['b1'].reshape(1, H2), p['g1'].reshape(1, H2),
      p['be1'].reshape(1, H2), p['W2'], p['b2'].reshape(1, D),
      lng.reshape(1, D), lnb.reshape(1, D), t.reshape(1, 1))


def _head_body(xn_ref, h1_ref, h2_ref, h3_ref, phi_ref, phib_ref,
               wa_ref, ba_ref, wb_ref, bb_ref, wc_ref, bc_ref,
               rho_ref, rhob_ref, cls_ref, clsb_ref, out_ref,
               num_ref, den_ref):
    i = pl.program_id(0)

    @pl.when(i == 0)
    def _init():
        num_ref[...] = jnp.zeros_like(num_ref)
        den_ref[...] = jnp.zeros_like(den_ref)

    phi = phi_ref[...]
    hp = (_bdot(xn_ref[...], phi[0:D])
          + _bdot(h1_ref[...], phi[D:2 * D])
          + _bdot(h2_ref[...], phi[2 * D:3 * D])
          + _bdot(h3_ref[...], phi[3 * D:4 * D])
          + phib_ref[...])
    hp = jnp.maximum(hp, 0.0)
    a = jnp.tanh(_bdot(hp, wa_ref[...])
                 + ba_ref[...])
    b = jax.nn.sigmoid(_bdot(hp, wb_ref[...])
                       + bb_ref[...])
    A = jnp.sum((a * b) * wc_ref[...], axis=1, keepdims=True) + bc_ref[0, 0]
    w = jnp.exp(A)
    num_ref[...] += jnp.sum(w * hp, axis=0, keepdims=True)
    den_ref[...] += jnp.sum(w)

    @pl.when(i == NSTEP - 1)
    def _final():
        pooled = num_ref[...] / den_ref[0, 0]
        hout = jnp.maximum(
            _bdot(pooled, rho_ref[...])
            + rhob_ref[...], 0.0)
        logit = jnp.sum(hout * cls_ref[...]) + clsb_ref[0, 0]
        out_ref[...] = jnp.full((1, 1), logit, jnp.float32)


def _run_head(xn, h1, h2, h3, params):
    full = lambda shp: pl.BlockSpec(shp, lambda i: tuple(0 for _ in shp))
    rb = pl.BlockSpec((RB, D), lambda i: (i, 0))
    return pl.pallas_call(
        _head_body,
        grid=(NSTEP,),
        in_specs=[
            rb, rb, rb, rb,
            full((H4, H4)), full((1, H4)),
            full((H4, H4)), full((1, H4)),
            full((H4, H4)), full((1, H4)),
            full((1, H4)), full((1, 1)),
            full((H4, H4)), full((1, H4)),
            full((1, H4)), full((1, 1)),
        ],
        out_specs=pl.BlockSpec((1, 1), lambda i: (0, 0)),
        out_shape=jax.ShapeDtypeStruct((1, 1), jnp.float32),
        scratch_shapes=[
            pltpu.VMEM((1, H4), jnp.float32),
            pltpu.VMEM((1, 1), jnp.float32),
        ],
    )(xn, h1, h2, h3,
      params['phi_W'], params['phi_b'].reshape(1, H4),
      params['Wa'], params['ba'].reshape(1, H4),
      params['Wb'], params['bb'].reshape(1, H4),
      params['Wc'].reshape(1, H4), params['bc'].reshape(1, 1),
      params['rho_W'], params['rho_b'].reshape(1, H4),
      params['cls_W'].reshape(1, H4), params['cls_b'].reshape(1, 1))


SUBS = (40, 39)


def _agg_body(tab_ref, srcidx_ref, dstidx_ref, out_ref,
              acc_ref, srcslab_ref, dstslab_ref, gbuf_ref, gsem):
    c = lax.axis_index("c")
    s = lax.axis_index("s")
    base = s * STRIPE

    def _zero_stripe():
        def _z(k, _):
            gbuf_ref[0, k // 8, pl.ds((k % 8) * 16, 16)] = jnp.zeros(
                (16,), jnp.float32)
            return _
        lax.fori_loop(0, CHUNK * 8, _z, None)
        for k in range(STRIPE // CHUNK):
            pltpu.sync_copy(gbuf_ref.at[0],
                            acc_ref.at[pl.ds(base + CHUNK * k, CHUNK)])
        rem = STRIPE % CHUNK
        pltpu.sync_copy(gbuf_ref.at[0, pl.ds(0, rem)],
                        acc_ref.at[pl.ds(base + STRIPE - rem, rem)])

    _zero_stripe()
    plsc.subcore_barrier()

    for bi in range(2):
        b = 2 * c + bi
        for sub, nch in enumerate(SUBS):
            h0 = sub * SUBS[0]
            pltpu.sync_copy(srcidx_ref.at[b, s, pl.ds(h0, nch)],
                            srcslab_ref.at[pl.ds(0, nch)])
            pltpu.sync_copy(dstidx_ref.at[s, pl.ds(h0, nch)],
                            dstslab_ref.at[pl.ds(0, nch)])
            for k in range(NBUF - 1):
                pltpu.async_copy(tab_ref.at[srcslab_ref.at[k]],
                                 gbuf_ref.at[k], gsem)

            def _chunk(k, _):
                j = lax.rem(k, NBUF)

                @pl.when(k < nch - (NBUF - 1))
                def _():
                    pltpu.async_copy(
                        tab_ref.at[srcslab_ref.at[k + NBUF - 1]],
                        gbuf_ref.at[lax.rem(k + NBUF - 1, NBUF)], gsem)
                pltpu.make_async_copy(tab_ref.at[pl.ds(0, CHUNK)],
                                      gbuf_ref.at[j], gsem).wait()
                pltpu.sync_copy(gbuf_ref.at[j],
                                acc_ref.at[dstslab_ref.at[k]], add=True)
                return _
            lax.fori_loop(0, nch, _chunk, None)
        plsc.subcore_barrier()

        pltpu.sync_copy(acc_ref.at[pl.ds(base, STRIPE)],
                        out_ref.at[b, pl.ds(base, STRIPE)])
        if bi == 0:
            _zero_stripe()
            plsc.subcore_barrier()


def _run_agg(tabflat, srcidx, dstidx):
    mesh = plsc.VectorSubcoreMesh(core_axis_name="c", subcore_axis_name="s")
    return pl.kernel(
        _agg_body,
        out_type=jax.ShapeDtypeStruct((4, NP, 128), jnp.float32),
        mesh=mesh,
        scratch_types=[
            pltpu.VMEM_SHARED((NP, 128), jnp.float32),
            pltpu.VMEM((SUBS[0], CHUNK), jnp.int32),
            pltpu.VMEM((SUBS[0], CHUNK), jnp.int32),
            pltpu.VMEM((NBUF, CHUNK, 128), jnp.float32),
            pltpu.SemaphoreType.DMA,
        ],
    )(tabflat, srcidx, dstidx)


PAD_PER = EPT - NE // 16


def _edge_slabs(edge_index):
    src = edge_index[0].astype(jnp.int32).reshape(16, NE // 16)
    dst = edge_index[1].astype(jnp.int32).reshape(16, NE // 16)
    pad_src = jnp.broadcast_to(jnp.arange(PAD_PER, dtype=jnp.int32) % 997,
                               (16, PAD_PER))
    pad_dst = jnp.broadcast_to(
        N + jnp.arange(PAD_PER, dtype=jnp.int32) % (NP - N), (16, PAD_PER))
    src_p = jnp.concatenate([src, pad_src], axis=1)
    dst_p = jnp.concatenate([dst, pad_dst], axis=1)
    offs = (jnp.arange(4, dtype=jnp.int32) * N)[:, None, None]
    srcidx = (src_p[None] + offs).reshape(4, 16, NCHUNK, CHUNK)
    dstidx = dst_p.reshape(16, NCHUNK, CHUNK)
    return srcidx, dstidx


def kernel(fea_old, x, edge_index, edge_attr, params):
    L = params['layers']
    srcidx, dstidx = _edge_slabs(edge_index)
    xn, tab = _run_fc(x, params['fc_W'], params['fc_b'], L[0]['t'])
    acc = _run_agg(tab.reshape(4 * N, 128), srcidx, dstidx)[:, :N, :]
    h1, tab = _run_mlp(acc, xn, L[0], False, L[1]['t'])
    acc = _run_agg(tab.reshape(4 * N, 128), srcidx, dstidx)[:, :N, :]
    h2, tab = _run_mlp(acc, h1, L[1], True, L[2]['t'])
    acc = _run_agg(tab.reshape(4 * N, 128), srcidx, dstidx)[:, :N, :]
    (h3,) = _run_mlp(acc, h2, L[2], True, None)
    return _run_head(xn, h1, h2, h3, params)
